# baseline (speedup 1.0000x reference)
import functools

import jax
import jax.numpy as jnp
from jax import lax
from jax.experimental import pallas as pl
from jax.experimental.pallas import tpu as pltpu
from jax.experimental.pallas import tpu_sc as plsc

B = 16384
N_WORD_FEAT = 18
N_TAG_FEAT = 18
N_DEPREL_FEAT = 12
EMBED = 64
HIDDEN = 200
N_CLASSES = 80

NC = 2
NS = 16
NW = NC * NS
L = 16

NPAIR = N_WORD_FEAT // 2
N_WROWS = NPAIR * B
CHUNK = 128
N_CHUNKS = N_WROWS // CHUNK
CHUNKS_PER_W = N_CHUNKS // NW
CHUNKS_PER_J = B // CHUNK


def _sc_gather(table, wi3):
    mesh = plsc.VectorSubcoreMesh(
        core_axis_name="c", subcore_axis_name="s", num_cores=NC, num_subcores=NS
    )

    @functools.partial(
        pl.kernel,
        out_type=jax.ShapeDtypeStruct((NPAIR, B, 2 * EMBED), jnp.bfloat16),
        mesh=mesh,
        scratch_types=[
            pltpu.VMEM((CHUNK, N_WORD_FEAT), jnp.int32),
            pltpu.VMEM((CHUNK, N_WORD_FEAT), jnp.int32),
            pltpu.VMEM((2, CHUNK), jnp.int32),
            pltpu.VMEM((2, CHUNK), jnp.int32),
            pltpu.VMEM((CHUNK, EMBED), jnp.bfloat16),
            pltpu.VMEM((CHUNK, EMBED), jnp.bfloat16),
            pltpu.VMEM((CHUNK, EMBED), jnp.bfloat16),
            pltpu.VMEM((CHUNK, EMBED), jnp.bfloat16),
            pltpu.SemaphoreType.DMA,
            pltpu.SemaphoreType.DMA,
            pltpu.SemaphoreType.DMA,
            pltpu.SemaphoreType.DMA,
        ],
        compiler_params=pltpu.CompilerParams(
            use_tc_tiling_on_sc=False, needs_layout_passes=False),
    )
    def gather_kernel(table_hbm, wi3_hbm, out_hbm,
                      ids0, ids1, idxe, idxo,
                      rowse0, rowso0, rowse1, rowso1,
                      gsem0, gsem1, isem0, isem1):
        wid = lax.axis_index("s") * NC + lax.axis_index("c")
        chunk_base = wid * CHUNKS_PER_W

        ids = (ids0, ids1)
        rows_e = (rowse0, rowse1)
        rows_o = (rowso0, rowso1)
        gsems = (gsem0, gsem1)
        isems = (isem0, isem1)

        def chunk_coords(c):
            jj = lax.div(c, CHUNKS_PER_J)
            b0 = lax.rem(c, CHUNKS_PER_J) * CHUNK
            return jj, b0

        def fire_ids(j, par):
            _, b0 = chunk_coords(chunk_base + j)
            pltpu.async_copy(wi3_hbm.at[pl.ds(b0, CHUNK)], ids[par], isems[par])

        def wait_ids(j, par):
            _, b0 = chunk_coords(chunk_base + j)
            pltpu.make_async_copy(
                wi3_hbm.at[pl.ds(b0, CHUNK)], ids[par], isems[par]).wait()

        def extract_idx(j, par):
            jj, _ = chunk_coords(chunk_base + j)
            for p, dst in ((0, idxe), (1, idxo)):
                cvec = jnp.full((L,), p, jnp.int32) + 2 * jj
                for k in range(CHUNK // L):
                    bvec = jnp.arange(k * L, (k + 1) * L, dtype=jnp.int32)
                    v = plsc.load_gather(ids[par], [bvec, cvec])
                    dst[par, pl.ds(k * L, L)] = v

        def fire_gather(j, par):
            pltpu.async_copy(table_hbm.at[idxe.at[par]], rows_e[par], gsems[par])
            pltpu.async_copy(table_hbm.at[idxo.at[par]], rows_o[par], gsems[par])

        def drain_gather_and_write(j, par):
            pltpu.make_async_copy(
                table_hbm.at[idxe.at[par]], rows_e[par], gsems[par]).wait()
            pltpu.make_async_copy(
                table_hbm.at[idxo.at[par]], rows_o[par], gsems[par]).wait()
            jj, b0 = chunk_coords(chunk_base + j)
            pltpu.sync_copy(
                rows_e[par], out_hbm.at[jj, pl.ds(b0, CHUNK), pl.ds(0, EMBED)])
            pltpu.sync_copy(
                rows_o[par],
                out_hbm.at[jj, pl.ds(b0, CHUNK), pl.ds(EMBED, EMBED)])

        pltpu.sync_copy(
            wi3_hbm.at[pl.ds(chunk_coords(chunk_base)[1], CHUNK)], ids0)
        extract_idx(0, 0)
        fire_gather(0, 0)
        fire_ids(1, 1)

        def body(j, _):
            for par in range(2):
                @pl.when(lax.rem(j, 2) == par)
                def _():
                    nxt = 1 - par
                    @pl.when(j + 1 < CHUNKS_PER_W)
                    def _prep():
                        wait_ids(j + 1, nxt)
                        extract_idx(j + 1, nxt)
                        fire_gather(j + 1, nxt)

                        @pl.when(j + 2 < CHUNKS_PER_W)
                        def _pref():
                            fire_ids(j + 2, par)

                    drain_gather_and_write(j, par)

            return 0

        lax.fori_loop(0, CHUNKS_PER_W, body, 0)

    return gather_kernel(table, wi3)


def _mlp_body(g2_ref, tag_ref, dep_ref, temb_ref, demb_ref, w1_ref, b1_ref,
              w2_ref, b2_ref, out_ref, pt_ref, pd_ref):
    blk = tag_ref.shape[0]

    @pl.when(pl.program_id(0) == 0)
    def _build_proj():
        for f in range(N_TAG_FEAT):
            base = N_WORD_FEAT * EMBED + f * EMBED
            pt_ref[f * EMBED:(f + 1) * EMBED, :] = jnp.dot(
                temb_ref[...], w1_ref[base:base + EMBED, :],
                preferred_element_type=jnp.float32)
        for f in range(N_DEPREL_FEAT):
            base = (N_WORD_FEAT + N_TAG_FEAT) * EMBED + f * EMBED
            pd_ref[f * EMBED:(f + 1) * EMBED, :] = jnp.dot(
                demb_ref[...], w1_ref[base:base + EMBED, :],
                preferred_element_type=jnp.float32)

    h = jnp.dot(g2_ref[0], w1_ref[0:2 * EMBED, :].astype(jnp.bfloat16),
                preferred_element_type=jnp.float32)
    for j in range(1, NPAIR):
        h = h + jnp.dot(
            g2_ref[j],
            w1_ref[j * 2 * EMBED:(j + 1) * 2 * EMBED, :].astype(jnp.bfloat16),
            preferred_element_type=jnp.float32)

    tag_ids = tag_ref[...]
    dep_ids = dep_ref[...]
    a_t = jnp.concatenate(
        [jnp.broadcast_to(tag_ids[:, f:f + 1], (blk, EMBED))
         for f in range(N_TAG_FEAT)], axis=1)
    a_d = jnp.concatenate(
        [jnp.broadcast_to(dep_ids[:, f:f + 1], (blk, EMBED))
         for f in range(N_DEPREL_FEAT)], axis=1)
    t_t = lax.rem(lax.broadcasted_iota(jnp.int32, (blk, N_TAG_FEAT * EMBED), 1),
                  EMBED)
    t_d = lax.rem(lax.broadcasted_iota(jnp.int32, (blk, N_DEPREL_FEAT * EMBED), 1),
                  EMBED)
    oh_t = (a_t == t_t).astype(jnp.float32)
    oh_d = (a_d == t_d).astype(jnp.float32)

    h = h + jnp.dot(oh_t, pt_ref[...], preferred_element_type=jnp.float32)
    h = h + jnp.dot(oh_d, pd_ref[...], preferred_element_type=jnp.float32)
    h = jnp.maximum(h + b1_ref[...], 0.0)
    out_ref[...] = jnp.dot(h, w2_ref[...],
                           preferred_element_type=jnp.float32) + b2_ref[...]


def _mlp(g2, tag_ids, dep_ids, tag_emb, deprel_emb, W1, b1, W2, b2):
    blk = 512
    grid = (B // blk,)
    return pl.pallas_call(
        _mlp_body,
        grid=grid,
        in_specs=[
            pl.BlockSpec((NPAIR, blk, 2 * EMBED), lambda i: (0, i, 0)),
            pl.BlockSpec((blk, N_TAG_FEAT), lambda i: (i, 0)),
            pl.BlockSpec((blk, N_DEPREL_FEAT), lambda i: (i, 0)),
            pl.BlockSpec((EMBED, EMBED), lambda i: (0, 0)),
            pl.BlockSpec((EMBED, EMBED), lambda i: (0, 0)),
            pl.BlockSpec((W1.shape[0], HIDDEN), lambda i: (0, 0)),
            pl.BlockSpec((1, HIDDEN), lambda i: (0, 0)),
            pl.BlockSpec((HIDDEN, N_CLASSES), lambda i: (0, 0)),
            pl.BlockSpec((1, N_CLASSES), lambda i: (0, 0)),
        ],
        out_specs=pl.BlockSpec((blk, N_CLASSES), lambda i: (i, 0)),
        out_shape=jax.ShapeDtypeStruct((B, N_CLASSES), jnp.float32),
        scratch_shapes=[
            pltpu.VMEM((N_TAG_FEAT * EMBED, HIDDEN), jnp.float32),
            pltpu.VMEM((N_DEPREL_FEAT * EMBED, HIDDEN), jnp.float32),
        ],
    )(g2, tag_ids, dep_ids, tag_emb, deprel_emb, W1, b1, W2, b2)


def kernel(word_id_batch, tag_id_batch, deprel_id_batch, word_emb, tag_emb,
           deprel_emb, W1, b1, W2, b2):
    g2 = _sc_gather(word_emb.astype(jnp.bfloat16), word_id_batch)
    return _mlp(g2, tag_id_batch, deprel_id_batch, tag_emb, deprel_emb,
                W1, b1.reshape(1, HIDDEN), W2, b2.reshape(1, N_CLASSES))


# speedup vs baseline: 2.1773x; 2.1773x over previous
import functools

import jax
import jax.numpy as jnp
from jax import lax
from jax.experimental import pallas as pl
from jax.experimental.pallas import tpu as pltpu
from jax.experimental.pallas import tpu_sc as plsc

B = 16384
N_WORD_FEAT = 18
N_TAG_FEAT = 18
N_DEPREL_FEAT = 12
EMBED = 64
HIDDEN = 200
N_CLASSES = 80

NC = 2
NS = 16
NW = NC * NS
L = 16

NPAIR = N_WORD_FEAT // 2
N_WROWS = NPAIR * B
CHUNK = 128
N_CHUNKS = N_WROWS // CHUNK
CHUNKS_PER_W = N_CHUNKS // NW
CHUNKS_PER_J = B // CHUNK


def _sc_gather(table, wi3):
    mesh = plsc.VectorSubcoreMesh(
        core_axis_name="c", subcore_axis_name="s", num_cores=NC, num_subcores=NS
    )

    @functools.partial(
        pl.kernel,
        out_type=jax.ShapeDtypeStruct((NPAIR, B, 2 * EMBED), jnp.float32),
        mesh=mesh,
        scratch_types=[
            pltpu.VMEM((CHUNK, N_WORD_FEAT), jnp.int32),
            pltpu.VMEM((CHUNK, N_WORD_FEAT), jnp.int32),
            pltpu.VMEM((2, CHUNK), jnp.int32),
            pltpu.VMEM((2, CHUNK), jnp.int32),
            pltpu.VMEM((CHUNK, EMBED), jnp.float32),
            pltpu.VMEM((CHUNK, EMBED), jnp.float32),
            pltpu.VMEM((CHUNK, EMBED), jnp.float32),
            pltpu.VMEM((CHUNK, EMBED), jnp.float32),
            pltpu.SemaphoreType.DMA,
            pltpu.SemaphoreType.DMA,
            pltpu.SemaphoreType.DMA,
            pltpu.SemaphoreType.DMA,
        ],
        compiler_params=pltpu.CompilerParams(
            use_tc_tiling_on_sc=False, needs_layout_passes=False),
    )
    def gather_kernel(table_hbm, wi3_hbm, out_hbm,
                      ids0, ids1, idxe, idxo,
                      rowse0, rowso0, rowse1, rowso1,
                      gsem0, gsem1, isem0, isem1):
        wid = lax.axis_index("s") * NC + lax.axis_index("c")
        chunk_base = wid * CHUNKS_PER_W

        ids = (ids0, ids1)
        rows_e = (rowse0, rowse1)
        rows_o = (rowso0, rowso1)
        gsems = (gsem0, gsem1)
        isems = (isem0, isem1)

        def chunk_coords(c):
            jj = lax.div(c, CHUNKS_PER_J)
            b0 = lax.rem(c, CHUNKS_PER_J) * CHUNK
            return jj, b0

        def fire_ids(j, par):
            _, b0 = chunk_coords(chunk_base + j)
            pltpu.async_copy(wi3_hbm.at[pl.ds(b0, CHUNK)], ids[par], isems[par])

        def wait_ids(j, par):
            _, b0 = chunk_coords(chunk_base + j)
            pltpu.make_async_copy(
                wi3_hbm.at[pl.ds(b0, CHUNK)], ids[par], isems[par]).wait()

        def extract_idx(j, par):
            jj, _ = chunk_coords(chunk_base + j)
            for p, dst in ((0, idxe), (1, idxo)):
                cvec = jnp.full((L,), p, jnp.int32) + 2 * jj
                for k in range(CHUNK // L):
                    bvec = jnp.arange(k * L, (k + 1) * L, dtype=jnp.int32)
                    v = plsc.load_gather(ids[par], [bvec, cvec])
                    q = jnp.bitwise_and(v, NB - 1)
                    r = v + q - jnp.where(q < NB // 2, 0, NB - 1)
                    dst[par, pl.ds(k * L, L)] = r

        def fire_gather(j, par):
            pltpu.async_copy(table_hbm.at[idxe.at[par]], rows_e[par], gsems[par])
            pltpu.async_copy(table_hbm.at[idxo.at[par]], rows_o[par], gsems[par])

        def drain_gather_and_write(j, par):
            pltpu.make_async_copy(
                table_hbm.at[idxe.at[par]], rows_e[par], gsems[par]).wait()
            pltpu.make_async_copy(
                table_hbm.at[idxo.at[par]], rows_o[par], gsems[par]).wait()
            jj, b0 = chunk_coords(chunk_base + j)
            pltpu.sync_copy(
                rows_e[par], out_hbm.at[jj, pl.ds(b0, CHUNK), pl.ds(0, EMBED)])
            pltpu.sync_copy(
                rows_o[par],
                out_hbm.at[jj, pl.ds(b0, CHUNK), pl.ds(EMBED, EMBED)])

        pltpu.sync_copy(
            wi3_hbm.at[pl.ds(chunk_coords(chunk_base)[1], CHUNK)], ids0)
        extract_idx(0, 0)
        fire_gather(0, 0)
        fire_ids(1, 1)

        def body(j, _):
            for par in range(2):
                @pl.when(lax.rem(j, 2) == par)
                def _():
                    nxt = 1 - par
                    @pl.when(j + 1 < CHUNKS_PER_W)
                    def _prep():
                        wait_ids(j + 1, nxt)
                        extract_idx(j + 1, nxt)
                        fire_gather(j + 1, nxt)

                        @pl.when(j + 2 < CHUNKS_PER_W)
                        def _pref():
                            fire_ids(j + 2, par)

                    drain_gather_and_write(j, par)

            return 0

        lax.fori_loop(0, CHUNKS_PER_W, body, 0)

    return gather_kernel(table, wi3)


NB = 4096


def _detile_body(xt_ref, out_ref):
    x = xt_ref[...]
    left = jnp.transpose(x[:, 0:NB // 2])
    right = jnp.transpose(x[:, NB // 2:NB])
    out_ref[...] = jnp.concatenate([left, right], axis=1)


def _detile(tableT):
    n_words = tableT.shape[1]
    n_blocks = (n_words + NB - 1) // NB
    return pl.pallas_call(
        _detile_body,
        grid=(n_blocks,),
        in_specs=[pl.BlockSpec((EMBED, NB), lambda i: (0, i))],
        out_specs=pl.BlockSpec((NB // 2, 2 * EMBED), lambda i: (i, 0)),
        out_shape=jax.ShapeDtypeStruct(
            (n_blocks * (NB // 2), 2 * EMBED), jnp.float32),
    )(tableT)


def _mlp_body(g2_ref, tag_ref, dep_ref, temb_ref, demb_ref, w1_ref, b1_ref,
              w2_ref, b2_ref, out_ref, pt_ref, pd_ref):
    blk = tag_ref.shape[0]

    @pl.when(pl.program_id(0) == 0)
    def _build_proj():
        for f in range(N_TAG_FEAT):
            base = N_WORD_FEAT * EMBED + f * EMBED
            pt_ref[f * EMBED:(f + 1) * EMBED, :] = jnp.dot(
                temb_ref[...], w1_ref[base:base + EMBED, :],
                preferred_element_type=jnp.float32)
        for f in range(N_DEPREL_FEAT):
            base = (N_WORD_FEAT + N_TAG_FEAT) * EMBED + f * EMBED
            pd_ref[f * EMBED:(f + 1) * EMBED, :] = jnp.dot(
                demb_ref[...], w1_ref[base:base + EMBED, :],
                preferred_element_type=jnp.float32)

    h = jnp.dot(g2_ref[0], w1_ref[0:2 * EMBED, :],
                preferred_element_type=jnp.float32)
    for j in range(1, NPAIR):
        h = h + jnp.dot(g2_ref[j], w1_ref[j * 2 * EMBED:(j + 1) * 2 * EMBED, :],
                        preferred_element_type=jnp.float32)

    tag_ids = tag_ref[...]
    dep_ids = dep_ref[...]
    a_t = jnp.concatenate(
        [jnp.broadcast_to(tag_ids[:, f:f + 1], (blk, EMBED))
         for f in range(N_TAG_FEAT)], axis=1)
    a_d = jnp.concatenate(
        [jnp.broadcast_to(dep_ids[:, f:f + 1], (blk, EMBED))
         for f in range(N_DEPREL_FEAT)], axis=1)
    t_t = lax.rem(lax.broadcasted_iota(jnp.int32, (blk, N_TAG_FEAT * EMBED), 1),
                  EMBED)
    t_d = lax.rem(lax.broadcasted_iota(jnp.int32, (blk, N_DEPREL_FEAT * EMBED), 1),
                  EMBED)
    oh_t = (a_t == t_t).astype(jnp.float32)
    oh_d = (a_d == t_d).astype(jnp.float32)

    h = h + jnp.dot(oh_t, pt_ref[...], preferred_element_type=jnp.float32)
    h = h + jnp.dot(oh_d, pd_ref[...], preferred_element_type=jnp.float32)
    h = jnp.maximum(h + b1_ref[...], 0.0)
    out_ref[...] = jnp.dot(h, w2_ref[...],
                           preferred_element_type=jnp.float32) + b2_ref[...]


def _mlp(g2, tag_ids, dep_ids, tag_emb, deprel_emb, W1, b1, W2, b2):
    blk = 512
    grid = (B // blk,)
    return pl.pallas_call(
        _mlp_body,
        grid=grid,
        in_specs=[
            pl.BlockSpec((NPAIR, blk, 2 * EMBED), lambda i: (0, i, 0)),
            pl.BlockSpec((blk, N_TAG_FEAT), lambda i: (i, 0)),
            pl.BlockSpec((blk, N_DEPREL_FEAT), lambda i: (i, 0)),
            pl.BlockSpec((EMBED, EMBED), lambda i: (0, 0)),
            pl.BlockSpec((EMBED, EMBED), lambda i: (0, 0)),
            pl.BlockSpec((W1.shape[0], HIDDEN), lambda i: (0, 0)),
            pl.BlockSpec((1, HIDDEN), lambda i: (0, 0)),
            pl.BlockSpec((HIDDEN, N_CLASSES), lambda i: (0, 0)),
            pl.BlockSpec((1, N_CLASSES), lambda i: (0, 0)),
        ],
        out_specs=pl.BlockSpec((blk, N_CLASSES), lambda i: (i, 0)),
        out_shape=jax.ShapeDtypeStruct((B, N_CLASSES), jnp.float32),
        scratch_shapes=[
            pltpu.VMEM((N_TAG_FEAT * EMBED, HIDDEN), jnp.float32),
            pltpu.VMEM((N_DEPREL_FEAT * EMBED, HIDDEN), jnp.float32),
        ],
    )(g2, tag_ids, dep_ids, tag_emb, deprel_emb, W1, b1, W2, b2)


def kernel(word_id_batch, tag_id_batch, deprel_id_batch, word_emb, tag_emb,
           deprel_emb, W1, b1, W2, b2):
    table_lin = _detile(jnp.swapaxes(word_emb, 0, 1))
    table_sc = table_lin.reshape(table_lin.shape[0] * 2, EMBED)
    g2 = _sc_gather(table_sc, word_id_batch)
    return _mlp(g2, tag_id_batch, deprel_id_batch, tag_emb, deprel_emb,
                W1, b1.reshape(1, HIDDEN), W2, b2.reshape(1, N_CLASSES))


# speedup vs baseline: 2.5051x; 1.1505x over previous
import functools

import jax
import jax.numpy as jnp
from jax import lax
from jax.experimental import pallas as pl
from jax.experimental.pallas import tpu as pltpu
from jax.experimental.pallas import tpu_sc as plsc

B = 16384
N_WORD_FEAT = 18
N_TAG_FEAT = 18
N_DEPREL_FEAT = 12
EMBED = 64
HIDDEN = 200
N_CLASSES = 80

NC = 2
NS = 16
NW = NC * NS
L = 16

NPAIR = N_WORD_FEAT // 2
N_WROWS = NPAIR * B
CHUNK = 128
N_CHUNKS = N_WROWS // CHUNK
CHUNKS_PER_W = N_CHUNKS // NW
CHUNKS_PER_J = B // CHUNK


def _sc_gather(table, wi3):
    mesh = plsc.VectorSubcoreMesh(
        core_axis_name="c", subcore_axis_name="s", num_cores=NC, num_subcores=NS
    )

    @functools.partial(
        pl.kernel,
        out_type=jax.ShapeDtypeStruct((NPAIR, B, 2 * EMBED), jnp.float32),
        mesh=mesh,
        scratch_types=[
            pltpu.VMEM((CHUNK, N_WORD_FEAT), jnp.int32),
            pltpu.VMEM((CHUNK, N_WORD_FEAT), jnp.int32),
            pltpu.VMEM((2, CHUNK), jnp.int32),
            pltpu.VMEM((2, CHUNK), jnp.int32),
            pltpu.VMEM((CHUNK, EMBED), jnp.float32),
            pltpu.VMEM((CHUNK, EMBED), jnp.float32),
            pltpu.VMEM((CHUNK, EMBED), jnp.float32),
            pltpu.VMEM((CHUNK, EMBED), jnp.float32),
            pltpu.SemaphoreType.DMA,
            pltpu.SemaphoreType.DMA,
            pltpu.SemaphoreType.DMA,
            pltpu.SemaphoreType.DMA,
        ],
        compiler_params=pltpu.CompilerParams(
            use_tc_tiling_on_sc=False, needs_layout_passes=False),
    )
    def gather_kernel(table_hbm, wi3_hbm, out_hbm,
                      ids0, ids1, idxe, idxo,
                      rowse0, rowso0, rowse1, rowso1,
                      gsem0, gsem1, isem0, isem1):
        wid = lax.axis_index("s") * NC + lax.axis_index("c")
        chunk_base = wid * CHUNKS_PER_W

        ids = (ids0, ids1)
        rows_e = (rowse0, rowse1)
        rows_o = (rowso0, rowso1)
        gsems = (gsem0, gsem1)
        isems = (isem0, isem1)

        def chunk_coords(c):
            jj = lax.div(c, CHUNKS_PER_J)
            b0 = lax.rem(c, CHUNKS_PER_J) * CHUNK
            return jj, b0

        def fire_ids(j, par):
            _, b0 = chunk_coords(chunk_base + j)
            pltpu.async_copy(wi3_hbm.at[pl.ds(b0, CHUNK)], ids[par], isems[par])

        def wait_ids(j, par):
            _, b0 = chunk_coords(chunk_base + j)
            pltpu.make_async_copy(
                wi3_hbm.at[pl.ds(b0, CHUNK)], ids[par], isems[par]).wait()

        def extract_idx(j, par):
            jj, _ = chunk_coords(chunk_base + j)
            for p, dst in ((0, idxe), (1, idxo)):
                cvec = jnp.full((L,), p, jnp.int32) + 2 * jj
                for k in range(CHUNK // L):
                    bvec = jnp.arange(k * L, (k + 1) * L, dtype=jnp.int32)
                    v = plsc.load_gather(ids[par], [bvec, cvec])
                    q = jnp.bitwise_and(v, NB - 1)
                    r = v + q - jnp.where(q < NB // 2, 0, NB - 1)
                    dst[par, pl.ds(k * L, L)] = r

        def fire_gather(j, par):
            pltpu.async_copy(table_hbm.at[idxe.at[par]], rows_e[par], gsems[par])
            pltpu.async_copy(table_hbm.at[idxo.at[par]], rows_o[par], gsems[par])

        def drain_gather_and_write(j, par):
            pltpu.make_async_copy(
                table_hbm.at[idxe.at[par]], rows_e[par], gsems[par]).wait()
            pltpu.make_async_copy(
                table_hbm.at[idxo.at[par]], rows_o[par], gsems[par]).wait()
            jj, b0 = chunk_coords(chunk_base + j)
            pltpu.sync_copy(
                rows_e[par], out_hbm.at[jj, pl.ds(b0, CHUNK), pl.ds(0, EMBED)])
            pltpu.sync_copy(
                rows_o[par],
                out_hbm.at[jj, pl.ds(b0, CHUNK), pl.ds(EMBED, EMBED)])

        pltpu.sync_copy(
            wi3_hbm.at[pl.ds(chunk_coords(chunk_base)[1], CHUNK)], ids0)
        extract_idx(0, 0)
        fire_gather(0, 0)
        fire_ids(1, 1)

        def body(j, _):
            for par in range(2):
                @pl.when(lax.rem(j, 2) == par)
                def _():
                    nxt = 1 - par
                    @pl.when(j + 1 < CHUNKS_PER_W)
                    def _prep():
                        wait_ids(j + 1, nxt)
                        extract_idx(j + 1, nxt)
                        fire_gather(j + 1, nxt)

                        @pl.when(j + 2 < CHUNKS_PER_W)
                        def _pref():
                            fire_ids(j + 2, par)

                    drain_gather_and_write(j, par)

            return 0

        lax.fori_loop(0, CHUNKS_PER_W, body, 0)

    return gather_kernel(table, wi3)


NB = 8192


def _detile_body(xt_ref, out_ref):
    x = xt_ref[...]
    left = jnp.transpose(x[:, 0:NB // 2])
    right = jnp.transpose(x[:, NB // 2:NB])
    out_ref[...] = jnp.concatenate([left, right], axis=1)


def _detile(tableT):
    n_words = tableT.shape[1]
    n_blocks = (n_words + NB - 1) // NB
    return pl.pallas_call(
        _detile_body,
        grid=(n_blocks,),
        in_specs=[pl.BlockSpec((EMBED, NB), lambda i: (0, i))],
        out_specs=pl.BlockSpec((NB // 2, 2 * EMBED), lambda i: (i, 0)),
        out_shape=jax.ShapeDtypeStruct(
            (n_blocks * (NB // 2), 2 * EMBED), jnp.float32),
    )(tableT)


def _mlp_body(g2_ref, tag_ref, dep_ref, temb_ref, demb_ref, w1_ref, b1_ref,
              w2_ref, b2_ref, out_ref, pt_ref, pd_ref):
    blk = tag_ref.shape[0]

    @pl.when(pl.program_id(0) == 0)
    def _build_proj():
        for f in range(N_TAG_FEAT):
            base = N_WORD_FEAT * EMBED + f * EMBED
            pt_ref[f * EMBED:(f + 1) * EMBED, :] = jnp.dot(
                temb_ref[...], w1_ref[base:base + EMBED, :],
                preferred_element_type=jnp.float32)
        for f in range(N_DEPREL_FEAT):
            base = (N_WORD_FEAT + N_TAG_FEAT) * EMBED + f * EMBED
            pd_ref[f * EMBED:(f + 1) * EMBED, :] = jnp.dot(
                demb_ref[...], w1_ref[base:base + EMBED, :],
                preferred_element_type=jnp.float32)

    h = jnp.dot(g2_ref[0].astype(jnp.bfloat16),
                w1_ref[0:2 * EMBED, :].astype(jnp.bfloat16),
                preferred_element_type=jnp.float32)
    for j in range(1, NPAIR):
        h = h + jnp.dot(
            g2_ref[j].astype(jnp.bfloat16),
            w1_ref[j * 2 * EMBED:(j + 1) * 2 * EMBED, :].astype(jnp.bfloat16),
            preferred_element_type=jnp.float32)

    tag_ids = tag_ref[...]
    dep_ids = dep_ref[...]
    a_t = jnp.concatenate(
        [jnp.broadcast_to(tag_ids[:, f:f + 1], (blk, EMBED))
         for f in range(N_TAG_FEAT)], axis=1)
    a_d = jnp.concatenate(
        [jnp.broadcast_to(dep_ids[:, f:f + 1], (blk, EMBED))
         for f in range(N_DEPREL_FEAT)], axis=1)
    t_t = lax.rem(lax.broadcasted_iota(jnp.int32, (blk, N_TAG_FEAT * EMBED), 1),
                  EMBED)
    t_d = lax.rem(lax.broadcasted_iota(jnp.int32, (blk, N_DEPREL_FEAT * EMBED), 1),
                  EMBED)
    oh_t = (a_t == t_t).astype(jnp.bfloat16)
    oh_d = (a_d == t_d).astype(jnp.bfloat16)

    h = h + jnp.dot(oh_t, pt_ref[...].astype(jnp.bfloat16),
                    preferred_element_type=jnp.float32)
    h = h + jnp.dot(oh_d, pd_ref[...].astype(jnp.bfloat16),
                    preferred_element_type=jnp.float32)
    h = jnp.maximum(h + b1_ref[...], 0.0)
    out_ref[...] = jnp.dot(h, w2_ref[...],
                           preferred_element_type=jnp.float32) + b2_ref[...]


def _mlp(g2, tag_ids, dep_ids, tag_emb, deprel_emb, W1, b1, W2, b2):
    blk = 512
    grid = (B // blk,)
    return pl.pallas_call(
        _mlp_body,
        grid=grid,
        in_specs=[
            pl.BlockSpec((NPAIR, blk, 2 * EMBED), lambda i: (0, i, 0)),
            pl.BlockSpec((blk, N_TAG_FEAT), lambda i: (i, 0)),
            pl.BlockSpec((blk, N_DEPREL_FEAT), lambda i: (i, 0)),
            pl.BlockSpec((EMBED, EMBED), lambda i: (0, 0)),
            pl.BlockSpec((EMBED, EMBED), lambda i: (0, 0)),
            pl.BlockSpec((W1.shape[0], HIDDEN), lambda i: (0, 0)),
            pl.BlockSpec((1, HIDDEN), lambda i: (0, 0)),
            pl.BlockSpec((HIDDEN, N_CLASSES), lambda i: (0, 0)),
            pl.BlockSpec((1, N_CLASSES), lambda i: (0, 0)),
        ],
        out_specs=pl.BlockSpec((blk, N_CLASSES), lambda i: (i, 0)),
        out_shape=jax.ShapeDtypeStruct((B, N_CLASSES), jnp.float32),
        scratch_shapes=[
            pltpu.VMEM((N_TAG_FEAT * EMBED, HIDDEN), jnp.float32),
            pltpu.VMEM((N_DEPREL_FEAT * EMBED, HIDDEN), jnp.float32),
        ],
    )(g2, tag_ids, dep_ids, tag_emb, deprel_emb, W1, b1, W2, b2)


def kernel(word_id_batch, tag_id_batch, deprel_id_batch, word_emb, tag_emb,
           deprel_emb, W1, b1, W2, b2):
    table_lin = _detile(jnp.swapaxes(word_emb, 0, 1))
    table_sc = table_lin.reshape(table_lin.shape[0] * 2, EMBED)
    g2 = _sc_gather(table_sc, word_id_batch)
    return _mlp(g2, tag_id_batch, deprel_id_batch, tag_emb, deprel_emb,
                W1, b1.reshape(1, HIDDEN), W2, b2.reshape(1, N_CLASSES))


# speedup vs baseline: 2.5873x; 1.0328x over previous
import functools

import jax
import jax.numpy as jnp
from jax import lax
from jax.experimental import pallas as pl
from jax.experimental.pallas import tpu as pltpu
from jax.experimental.pallas import tpu_sc as plsc

B = 16384
N_WORD_FEAT = 18
N_TAG_FEAT = 18
N_DEPREL_FEAT = 12
EMBED = 64
HIDDEN = 200
N_CLASSES = 80

NC = 2
NS = 16
NW = NC * NS
L = 16

NPAIR = N_WORD_FEAT // 2
CHUNK = 128
N_SLICES = 2
BS = B // N_SLICES


def _sc_gather(table, wi3):
    Bs = wi3.shape[0]
    CHUNKS_PER_J = Bs // CHUNK
    CHUNKS_PER_W = (NPAIR * Bs // CHUNK) // NW
    mesh = plsc.VectorSubcoreMesh(
        core_axis_name="c", subcore_axis_name="s", num_cores=NC, num_subcores=NS
    )

    @functools.partial(
        pl.kernel,
        out_type=jax.ShapeDtypeStruct((NPAIR, Bs, 2 * EMBED), jnp.float32),
        mesh=mesh,
        scratch_types=[
            pltpu.VMEM((CHUNK, N_WORD_FEAT), jnp.int32),
            pltpu.VMEM((CHUNK, N_WORD_FEAT), jnp.int32),
            pltpu.VMEM((2, CHUNK), jnp.int32),
            pltpu.VMEM((2, CHUNK), jnp.int32),
            pltpu.VMEM((CHUNK, EMBED), jnp.float32),
            pltpu.VMEM((CHUNK, EMBED), jnp.float32),
            pltpu.VMEM((CHUNK, EMBED), jnp.float32),
            pltpu.VMEM((CHUNK, EMBED), jnp.float32),
            pltpu.SemaphoreType.DMA,
            pltpu.SemaphoreType.DMA,
            pltpu.SemaphoreType.DMA,
            pltpu.SemaphoreType.DMA,
        ],
        compiler_params=pltpu.CompilerParams(
            use_tc_tiling_on_sc=False, needs_layout_passes=False),
    )
    def gather_kernel(table_hbm, wi3_hbm, out_hbm,
                      ids0, ids1, idxe, idxo,
                      rowse0, rowso0, rowse1, rowso1,
                      gsem0, gsem1, isem0, isem1):
        wid = lax.axis_index("s") * NC + lax.axis_index("c")
        chunk_base = wid * CHUNKS_PER_W

        ids = (ids0, ids1)
        rows_e = (rowse0, rowse1)
        rows_o = (rowso0, rowso1)
        gsems = (gsem0, gsem1)
        isems = (isem0, isem1)

        def chunk_coords(c):
            jj = lax.div(c, CHUNKS_PER_J)
            b0 = lax.rem(c, CHUNKS_PER_J) * CHUNK
            return jj, b0

        def fire_ids(j, par):
            _, b0 = chunk_coords(chunk_base + j)
            pltpu.async_copy(wi3_hbm.at[pl.ds(b0, CHUNK)], ids[par], isems[par])

        def wait_ids(j, par):
            _, b0 = chunk_coords(chunk_base + j)
            pltpu.make_async_copy(
                wi3_hbm.at[pl.ds(b0, CHUNK)], ids[par], isems[par]).wait()

        def extract_idx(j, par):
            jj, _ = chunk_coords(chunk_base + j)
            for p, dst in ((0, idxe), (1, idxo)):
                cvec = jnp.full((L,), p, jnp.int32) + 2 * jj
                for k in range(CHUNK // L):
                    bvec = jnp.arange(k * L, (k + 1) * L, dtype=jnp.int32)
                    v = plsc.load_gather(ids[par], [bvec, cvec])
                    q = jnp.bitwise_and(v, NB - 1)
                    r = v + q - jnp.where(q < NB // 2, 0, NB - 1)
                    dst[par, pl.ds(k * L, L)] = r

        def fire_gather(j, par):
            pltpu.async_copy(table_hbm.at[idxe.at[par]], rows_e[par], gsems[par])
            pltpu.async_copy(table_hbm.at[idxo.at[par]], rows_o[par], gsems[par])

        def drain_gather_and_write(j, par):
            pltpu.make_async_copy(
                table_hbm.at[idxe.at[par]], rows_e[par], gsems[par]).wait()
            pltpu.make_async_copy(
                table_hbm.at[idxo.at[par]], rows_o[par], gsems[par]).wait()
            jj, b0 = chunk_coords(chunk_base + j)
            pltpu.sync_copy(
                rows_e[par], out_hbm.at[jj, pl.ds(b0, CHUNK), pl.ds(0, EMBED)])
            pltpu.sync_copy(
                rows_o[par],
                out_hbm.at[jj, pl.ds(b0, CHUNK), pl.ds(EMBED, EMBED)])

        pltpu.sync_copy(
            wi3_hbm.at[pl.ds(chunk_coords(chunk_base)[1], CHUNK)], ids0)
        extract_idx(0, 0)
        fire_gather(0, 0)
        fire_ids(1, 1)

        def body(j, _):
            for par in range(2):
                @pl.when(lax.rem(j, 2) == par)
                def _():
                    nxt = 1 - par
                    @pl.when(j + 1 < CHUNKS_PER_W)
                    def _prep():
                        wait_ids(j + 1, nxt)
                        extract_idx(j + 1, nxt)
                        fire_gather(j + 1, nxt)

                        @pl.when(j + 2 < CHUNKS_PER_W)
                        def _pref():
                            fire_ids(j + 2, par)

                    drain_gather_and_write(j, par)

            return 0

        lax.fori_loop(0, CHUNKS_PER_W, body, 0)

    return gather_kernel(table, wi3)


NB = 8192


def _detile_body(xt_ref, out_ref):
    x = xt_ref[...]
    left = jnp.transpose(x[:, 0:NB // 2])
    right = jnp.transpose(x[:, NB // 2:NB])
    out_ref[...] = jnp.concatenate([left, right], axis=1)


def _detile(tableT):
    n_words = tableT.shape[1]
    n_blocks = (n_words + NB - 1) // NB
    return pl.pallas_call(
        _detile_body,
        grid=(n_blocks,),
        in_specs=[pl.BlockSpec((EMBED, NB), lambda i: (0, i))],
        out_specs=pl.BlockSpec((NB // 2, 2 * EMBED), lambda i: (i, 0)),
        out_shape=jax.ShapeDtypeStruct(
            (n_blocks * (NB // 2), 2 * EMBED), jnp.float32),
    )(tableT)


def _mlp_body(g2_ref, tag_ref, dep_ref, temb_ref, demb_ref, w1_ref, b1_ref,
              w2_ref, b2_ref, out_ref, pt_ref, pd_ref):
    blk = tag_ref.shape[0]

    @pl.when(pl.program_id(0) == 0)
    def _build_proj():
        for f in range(N_TAG_FEAT):
            base = N_WORD_FEAT * EMBED + f * EMBED
            pt_ref[f * EMBED:(f + 1) * EMBED, :] = jnp.dot(
                temb_ref[...], w1_ref[base:base + EMBED, :],
                preferred_element_type=jnp.float32)
        for f in range(N_DEPREL_FEAT):
            base = (N_WORD_FEAT + N_TAG_FEAT) * EMBED + f * EMBED
            pd_ref[f * EMBED:(f + 1) * EMBED, :] = jnp.dot(
                demb_ref[...], w1_ref[base:base + EMBED, :],
                preferred_element_type=jnp.float32)

    h = jnp.dot(g2_ref[0].astype(jnp.bfloat16),
                w1_ref[0:2 * EMBED, :].astype(jnp.bfloat16),
                preferred_element_type=jnp.float32)
    for j in range(1, NPAIR):
        h = h + jnp.dot(
            g2_ref[j].astype(jnp.bfloat16),
            w1_ref[j * 2 * EMBED:(j + 1) * 2 * EMBED, :].astype(jnp.bfloat16),
            preferred_element_type=jnp.float32)

    tag_ids = tag_ref[...]
    dep_ids = dep_ref[...]
    a_t = jnp.concatenate(
        [jnp.broadcast_to(tag_ids[:, f:f + 1], (blk, EMBED))
         for f in range(N_TAG_FEAT)], axis=1)
    a_d = jnp.concatenate(
        [jnp.broadcast_to(dep_ids[:, f:f + 1], (blk, EMBED))
         for f in range(N_DEPREL_FEAT)], axis=1)
    t_t = lax.rem(lax.broadcasted_iota(jnp.int32, (blk, N_TAG_FEAT * EMBED), 1),
                  EMBED)
    t_d = lax.rem(lax.broadcasted_iota(jnp.int32, (blk, N_DEPREL_FEAT * EMBED), 1),
                  EMBED)
    oh_t = (a_t == t_t).astype(jnp.bfloat16)
    oh_d = (a_d == t_d).astype(jnp.bfloat16)

    h = h + jnp.dot(oh_t, pt_ref[...].astype(jnp.bfloat16),
                    preferred_element_type=jnp.float32)
    h = h + jnp.dot(oh_d, pd_ref[...].astype(jnp.bfloat16),
                    preferred_element_type=jnp.float32)
    h = jnp.maximum(h + b1_ref[...], 0.0)
    out_ref[...] = jnp.dot(h, w2_ref[...],
                           preferred_element_type=jnp.float32) + b2_ref[...]


def _mlp(g2, tag_ids, dep_ids, tag_emb, deprel_emb, W1, b1, W2, b2):
    blk = 512
    grid = (tag_ids.shape[0] // blk,)
    return pl.pallas_call(
        _mlp_body,
        grid=grid,
        in_specs=[
            pl.BlockSpec((NPAIR, blk, 2 * EMBED), lambda i: (0, i, 0)),
            pl.BlockSpec((blk, N_TAG_FEAT), lambda i: (i, 0)),
            pl.BlockSpec((blk, N_DEPREL_FEAT), lambda i: (i, 0)),
            pl.BlockSpec((EMBED, EMBED), lambda i: (0, 0)),
            pl.BlockSpec((EMBED, EMBED), lambda i: (0, 0)),
            pl.BlockSpec((W1.shape[0], HIDDEN), lambda i: (0, 0)),
            pl.BlockSpec((1, HIDDEN), lambda i: (0, 0)),
            pl.BlockSpec((HIDDEN, N_CLASSES), lambda i: (0, 0)),
            pl.BlockSpec((1, N_CLASSES), lambda i: (0, 0)),
        ],
        out_specs=pl.BlockSpec((blk, N_CLASSES), lambda i: (i, 0)),
        out_shape=jax.ShapeDtypeStruct((tag_ids.shape[0], N_CLASSES),
                                       jnp.float32),
        scratch_shapes=[
            pltpu.VMEM((N_TAG_FEAT * EMBED, HIDDEN), jnp.float32),
            pltpu.VMEM((N_DEPREL_FEAT * EMBED, HIDDEN), jnp.float32),
        ],
    )(g2, tag_ids, dep_ids, tag_emb, deprel_emb, W1, b1, W2, b2)


def kernel(word_id_batch, tag_id_batch, deprel_id_batch, word_emb, tag_emb,
           deprel_emb, W1, b1, W2, b2):
    table_lin = _detile(jnp.swapaxes(word_emb, 0, 1))
    table_sc = table_lin.reshape(table_lin.shape[0] * 2, EMBED)
    b1r = b1.reshape(1, HIDDEN)
    b2r = b2.reshape(1, N_CLASSES)
    outs = []
    for s in range(N_SLICES):
        sl = slice(s * BS, (s + 1) * BS)
        g2 = _sc_gather(table_sc, word_id_batch[sl])
        outs.append(_mlp(g2, tag_id_batch[sl], deprel_id_batch[sl], tag_emb,
                         deprel_emb, W1, b1r, W2, b2r))
    return jnp.concatenate(outs, axis=0)


# speedup vs baseline: 2.7920x; 1.0791x over previous
import functools

import jax
import jax.numpy as jnp
from jax import lax
from jax.experimental import pallas as pl
from jax.experimental.pallas import tpu as pltpu
from jax.experimental.pallas import tpu_sc as plsc

B = 16384
N_WORD_FEAT = 18
N_TAG_FEAT = 18
N_DEPREL_FEAT = 12
EMBED = 64
HIDDEN = 200
N_CLASSES = 80

NC = 2
NS = 16
NW = NC * NS
L = 16

NPAIR = N_WORD_FEAT // 2
CHUNK = 128
N_SLICES = 2
BS = B // N_SLICES


def _sc_gather(table, wi3):
    Bs = wi3.shape[0]
    CHUNKS_PER_J = Bs // CHUNK
    CHUNKS_PER_W = (NPAIR * Bs // CHUNK) // NW
    mesh = plsc.VectorSubcoreMesh(
        core_axis_name="c", subcore_axis_name="s", num_cores=NC, num_subcores=NS
    )

    @functools.partial(
        pl.kernel,
        out_type=jax.ShapeDtypeStruct((NPAIR, Bs, 2 * EMBED), jnp.float32),
        mesh=mesh,
        scratch_types=[
            pltpu.VMEM((CHUNK, N_WORD_FEAT), jnp.int32),
            pltpu.VMEM((CHUNK, N_WORD_FEAT), jnp.int32),
            pltpu.VMEM((2, CHUNK), jnp.int32),
            pltpu.VMEM((2, CHUNK), jnp.int32),
            pltpu.VMEM((CHUNK, EMBED), jnp.float32),
            pltpu.VMEM((CHUNK, EMBED), jnp.float32),
            pltpu.VMEM((CHUNK, EMBED), jnp.float32),
            pltpu.VMEM((CHUNK, EMBED), jnp.float32),
            pltpu.SemaphoreType.DMA,
            pltpu.SemaphoreType.DMA,
            pltpu.SemaphoreType.DMA,
            pltpu.SemaphoreType.DMA,
        ],
        compiler_params=pltpu.CompilerParams(
            use_tc_tiling_on_sc=False, needs_layout_passes=False),
    )
    def gather_kernel(table_hbm, wi3_hbm, out_hbm,
                      ids0, ids1, idxe, idxo,
                      rowse0, rowso0, rowse1, rowso1,
                      gsem0, gsem1, isem0, isem1):
        wid = lax.axis_index("s") * NC + lax.axis_index("c")
        chunk_base = wid * CHUNKS_PER_W

        ids = (ids0, ids1)
        rows_e = (rowse0, rowse1)
        rows_o = (rowso0, rowso1)
        gsems = (gsem0, gsem1)
        isems = (isem0, isem1)

        def chunk_coords(c):
            jj = lax.div(c, CHUNKS_PER_J)
            b0 = lax.rem(c, CHUNKS_PER_J) * CHUNK
            return jj, b0

        def fire_ids(j, par):
            _, b0 = chunk_coords(chunk_base + j)
            pltpu.async_copy(wi3_hbm.at[pl.ds(b0, CHUNK)], ids[par], isems[par])

        def wait_ids(j, par):
            _, b0 = chunk_coords(chunk_base + j)
            pltpu.make_async_copy(
                wi3_hbm.at[pl.ds(b0, CHUNK)], ids[par], isems[par]).wait()

        def extract_idx(j, par):
            jj, _ = chunk_coords(chunk_base + j)
            for p, dst in ((0, idxe), (1, idxo)):
                cvec = jnp.full((L,), p, jnp.int32) + 2 * jj
                for k in range(CHUNK // L):
                    bvec = jnp.arange(k * L, (k + 1) * L, dtype=jnp.int32)
                    v = plsc.load_gather(ids[par], [bvec, cvec])
                    q = jnp.bitwise_and(v, NB - 1)
                    r = v + q - jnp.where(q < NB // 2, 0, NB - 1)
                    dst[par, pl.ds(k * L, L)] = r

        def fire_gather(j, par):
            pltpu.async_copy(table_hbm.at[idxe.at[par]], rows_e[par], gsems[par])
            pltpu.async_copy(table_hbm.at[idxo.at[par]], rows_o[par], gsems[par])

        def drain_gather_and_write(j, par):
            pltpu.make_async_copy(
                table_hbm.at[idxe.at[par]], rows_e[par], gsems[par]).wait()
            pltpu.make_async_copy(
                table_hbm.at[idxo.at[par]], rows_o[par], gsems[par]).wait()
            jj, b0 = chunk_coords(chunk_base + j)
            pltpu.sync_copy(
                rows_e[par], out_hbm.at[jj, pl.ds(b0, CHUNK), pl.ds(0, EMBED)])
            pltpu.sync_copy(
                rows_o[par],
                out_hbm.at[jj, pl.ds(b0, CHUNK), pl.ds(EMBED, EMBED)])

        pltpu.sync_copy(
            wi3_hbm.at[pl.ds(chunk_coords(chunk_base)[1], CHUNK)], ids0)
        extract_idx(0, 0)
        fire_gather(0, 0)
        fire_ids(1, 1)

        def body(j, _):
            for par in range(2):
                @pl.when(lax.rem(j, 2) == par)
                def _():
                    nxt = 1 - par
                    @pl.when(j + 1 < CHUNKS_PER_W)
                    def _prep():
                        wait_ids(j + 1, nxt)
                        extract_idx(j + 1, nxt)
                        fire_gather(j + 1, nxt)

                        @pl.when(j + 2 < CHUNKS_PER_W)
                        def _pref():
                            fire_ids(j + 2, par)

                    drain_gather_and_write(j, par)

            return 0

        lax.fori_loop(0, CHUNKS_PER_W, body, 0)

    return gather_kernel(table, wi3)


NB = 16384


def _detile_body(xt_ref, out_ref):
    x = xt_ref[...]
    left = jnp.transpose(x[:, 0:NB // 2])
    right = jnp.transpose(x[:, NB // 2:NB])
    out_ref[...] = jnp.concatenate([left, right], axis=1)


def _detile(tableT):
    n_words = tableT.shape[1]
    n_blocks = (n_words + NB - 1) // NB
    return pl.pallas_call(
        _detile_body,
        grid=(n_blocks,),
        in_specs=[pl.BlockSpec((EMBED, NB), lambda i: (0, i))],
        out_specs=pl.BlockSpec((NB // 2, 2 * EMBED), lambda i: (i, 0)),
        out_shape=jax.ShapeDtypeStruct(
            (n_blocks * (NB // 2), 2 * EMBED), jnp.float32),
    )(tableT)


def _mlp_body(g2_ref, tag_ref, dep_ref, temb_ref, demb_ref, w1_ref, b1_ref,
              w2_ref, b2_ref, out_ref, pt_ref, pd_ref):
    blk = tag_ref.shape[0]

    @pl.when(pl.program_id(0) == 0)
    def _build_proj():
        for f in range(N_TAG_FEAT):
            base = N_WORD_FEAT * EMBED + f * EMBED
            pt_ref[f * EMBED:(f + 1) * EMBED, :] = jnp.dot(
                temb_ref[...], w1_ref[base:base + EMBED, :],
                preferred_element_type=jnp.float32)
        for f in range(N_DEPREL_FEAT):
            base = (N_WORD_FEAT + N_TAG_FEAT) * EMBED + f * EMBED
            pd_ref[f * EMBED:(f + 1) * EMBED, :] = jnp.dot(
                demb_ref[...], w1_ref[base:base + EMBED, :],
                preferred_element_type=jnp.float32)

    h = jnp.dot(g2_ref[0].astype(jnp.bfloat16),
                w1_ref[0:2 * EMBED, :].astype(jnp.bfloat16),
                preferred_element_type=jnp.float32)
    for j in range(1, NPAIR):
        h = h + jnp.dot(
            g2_ref[j].astype(jnp.bfloat16),
            w1_ref[j * 2 * EMBED:(j + 1) * 2 * EMBED, :].astype(jnp.bfloat16),
            preferred_element_type=jnp.float32)

    tag_ids = tag_ref[...]
    dep_ids = dep_ref[...]
    a_t = jnp.concatenate(
        [jnp.broadcast_to(tag_ids[:, f:f + 1], (blk, EMBED))
         for f in range(N_TAG_FEAT)], axis=1)
    a_d = jnp.concatenate(
        [jnp.broadcast_to(dep_ids[:, f:f + 1], (blk, EMBED))
         for f in range(N_DEPREL_FEAT)], axis=1)
    t_t = lax.rem(lax.broadcasted_iota(jnp.int32, (blk, N_TAG_FEAT * EMBED), 1),
                  EMBED)
    t_d = lax.rem(lax.broadcasted_iota(jnp.int32, (blk, N_DEPREL_FEAT * EMBED), 1),
                  EMBED)
    oh_t = (a_t == t_t).astype(jnp.bfloat16)
    oh_d = (a_d == t_d).astype(jnp.bfloat16)

    h = h + jnp.dot(oh_t, pt_ref[...].astype(jnp.bfloat16),
                    preferred_element_type=jnp.float32)
    h = h + jnp.dot(oh_d, pd_ref[...].astype(jnp.bfloat16),
                    preferred_element_type=jnp.float32)
    h = jnp.maximum(h + b1_ref[...], 0.0)
    out_ref[...] = jnp.dot(h, w2_ref[...],
                           preferred_element_type=jnp.float32) + b2_ref[...]


def _mlp(g2, tag_ids, dep_ids, tag_emb, deprel_emb, W1, b1, W2, b2):
    blk = 512
    grid = (tag_ids.shape[0] // blk,)
    return pl.pallas_call(
        _mlp_body,
        grid=grid,
        in_specs=[
            pl.BlockSpec((NPAIR, blk, 2 * EMBED), lambda i: (0, i, 0)),
            pl.BlockSpec((blk, N_TAG_FEAT), lambda i: (i, 0)),
            pl.BlockSpec((blk, N_DEPREL_FEAT), lambda i: (i, 0)),
            pl.BlockSpec((EMBED, EMBED), lambda i: (0, 0)),
            pl.BlockSpec((EMBED, EMBED), lambda i: (0, 0)),
            pl.BlockSpec((W1.shape[0], HIDDEN), lambda i: (0, 0)),
            pl.BlockSpec((1, HIDDEN), lambda i: (0, 0)),
            pl.BlockSpec((HIDDEN, N_CLASSES), lambda i: (0, 0)),
            pl.BlockSpec((1, N_CLASSES), lambda i: (0, 0)),
        ],
        out_specs=pl.BlockSpec((blk, N_CLASSES), lambda i: (i, 0)),
        out_shape=jax.ShapeDtypeStruct((tag_ids.shape[0], N_CLASSES),
                                       jnp.float32),
        scratch_shapes=[
            pltpu.VMEM((N_TAG_FEAT * EMBED, HIDDEN), jnp.float32),
            pltpu.VMEM((N_DEPREL_FEAT * EMBED, HIDDEN), jnp.float32),
        ],
    )(g2, tag_ids, dep_ids, tag_emb, deprel_emb, W1, b1, W2, b2)


def kernel(word_id_batch, tag_id_batch, deprel_id_batch, word_emb, tag_emb,
           deprel_emb, W1, b1, W2, b2):
    table_lin = _detile(jnp.swapaxes(word_emb, 0, 1))
    table_sc = table_lin.reshape(table_lin.shape[0] * 2, EMBED)
    b1r = b1.reshape(1, HIDDEN)
    b2r = b2.reshape(1, N_CLASSES)
    outs = []
    for s in range(N_SLICES):
        sl = slice(s * BS, (s + 1) * BS)
        g2 = _sc_gather(table_sc, word_id_batch[sl])
        outs.append(_mlp(g2, tag_id_batch[sl], deprel_id_batch[sl], tag_emb,
                         deprel_emb, W1, b1r, W2, b2r))
    return jnp.concatenate(outs, axis=0)


# speedup vs baseline: 2.8993x; 1.0384x over previous
import functools

import jax
import jax.numpy as jnp
from jax import lax
from jax.experimental import pallas as pl
from jax.experimental.pallas import tpu as pltpu
from jax.experimental.pallas import tpu_sc as plsc

B = 16384
N_WORD_FEAT = 18
N_TAG_FEAT = 18
N_DEPREL_FEAT = 12
EMBED = 64
HIDDEN = 200
N_CLASSES = 80

NC = 2
NS = 16
NW = NC * NS
L = 16

NPAIR = N_WORD_FEAT // 2
CHUNK = 128
N_SLICES = 2
BS = B // N_SLICES


def _sc_gather(table, wi3):
    Bs = wi3.shape[0]
    CHUNKS_PER_J = Bs // CHUNK
    CHUNKS_PER_W = (NPAIR * Bs // CHUNK) // NW
    mesh = plsc.VectorSubcoreMesh(
        core_axis_name="c", subcore_axis_name="s", num_cores=NC, num_subcores=NS
    )

    @functools.partial(
        pl.kernel,
        out_type=jax.ShapeDtypeStruct((NPAIR, Bs, 2 * EMBED), jnp.float32),
        mesh=mesh,
        scratch_types=[
            pltpu.VMEM((CHUNK, N_WORD_FEAT), jnp.int32),
            pltpu.VMEM((CHUNK, N_WORD_FEAT), jnp.int32),
            pltpu.VMEM((2, CHUNK), jnp.int32),
            pltpu.VMEM((2, CHUNK), jnp.int32),
            pltpu.VMEM((CHUNK, EMBED), jnp.float32),
            pltpu.VMEM((CHUNK, EMBED), jnp.float32),
            pltpu.VMEM((CHUNK, EMBED), jnp.float32),
            pltpu.VMEM((CHUNK, EMBED), jnp.float32),
            pltpu.SemaphoreType.DMA,
            pltpu.SemaphoreType.DMA,
            pltpu.SemaphoreType.DMA,
            pltpu.SemaphoreType.DMA,
        ],
        compiler_params=pltpu.CompilerParams(
            use_tc_tiling_on_sc=False, needs_layout_passes=False),
    )
    def gather_kernel(table_hbm, wi3_hbm, out_hbm,
                      ids0, ids1, idxe, idxo,
                      rowse0, rowso0, rowse1, rowso1,
                      gsem0, gsem1, isem0, isem1):
        wid = lax.axis_index("s") * NC + lax.axis_index("c")
        chunk_base = wid * CHUNKS_PER_W

        ids = (ids0, ids1)
        rows_e = (rowse0, rowse1)
        rows_o = (rowso0, rowso1)
        gsems = (gsem0, gsem1)
        isems = (isem0, isem1)

        def chunk_coords(c):
            jj = lax.div(c, CHUNKS_PER_J)
            b0 = lax.rem(c, CHUNKS_PER_J) * CHUNK
            return jj, b0

        def fire_ids(j, par):
            _, b0 = chunk_coords(chunk_base + j)
            pltpu.async_copy(wi3_hbm.at[pl.ds(b0, CHUNK)], ids[par], isems[par])

        def wait_ids(j, par):
            _, b0 = chunk_coords(chunk_base + j)
            pltpu.make_async_copy(
                wi3_hbm.at[pl.ds(b0, CHUNK)], ids[par], isems[par]).wait()

        def extract_idx(j, par):
            jj, _ = chunk_coords(chunk_base + j)
            for p, dst in ((0, idxe), (1, idxo)):
                cvec = jnp.full((L,), p, jnp.int32) + 2 * jj
                for k in range(CHUNK // L):
                    bvec = jnp.arange(k * L, (k + 1) * L, dtype=jnp.int32)
                    v = plsc.load_gather(ids[par], [bvec, cvec])
                    q = jnp.bitwise_and(v, NB - 1)
                    r = v + q - jnp.where(q < NB // 2, 0, NB - 1)
                    dst[par, pl.ds(k * L, L)] = r

        def fire_gather(j, par):
            pltpu.async_copy(table_hbm.at[idxe.at[par]], rows_e[par], gsems[par])
            pltpu.async_copy(table_hbm.at[idxo.at[par]], rows_o[par], gsems[par])

        def drain_gather_and_write(j, par):
            pltpu.make_async_copy(
                table_hbm.at[idxe.at[par]], rows_e[par], gsems[par]).wait()
            pltpu.make_async_copy(
                table_hbm.at[idxo.at[par]], rows_o[par], gsems[par]).wait()
            jj, b0 = chunk_coords(chunk_base + j)
            pltpu.sync_copy(
                rows_e[par], out_hbm.at[jj, pl.ds(b0, CHUNK), pl.ds(0, EMBED)])
            pltpu.sync_copy(
                rows_o[par],
                out_hbm.at[jj, pl.ds(b0, CHUNK), pl.ds(EMBED, EMBED)])

        pltpu.sync_copy(
            wi3_hbm.at[pl.ds(chunk_coords(chunk_base)[1], CHUNK)], ids0)
        extract_idx(0, 0)
        fire_gather(0, 0)
        fire_ids(1, 1)

        def body(j, _):
            for par in range(2):
                @pl.when(lax.rem(j, 2) == par)
                def _():
                    nxt = 1 - par
                    @pl.when(j + 1 < CHUNKS_PER_W)
                    def _prep():
                        wait_ids(j + 1, nxt)
                        extract_idx(j + 1, nxt)
                        fire_gather(j + 1, nxt)

                        @pl.when(j + 2 < CHUNKS_PER_W)
                        def _pref():
                            fire_ids(j + 2, par)

                    drain_gather_and_write(j, par)

            return 0

        lax.fori_loop(0, CHUNKS_PER_W, body, 0)

    return gather_kernel(table, wi3)


NB = 32768


def _detile_body(xt_ref, out_ref):
    x = xt_ref[...]
    left = jnp.transpose(x[:, 0:NB // 2])
    right = jnp.transpose(x[:, NB // 2:NB])
    out_ref[...] = jnp.concatenate([left, right], axis=1)


def _detile(tableT):
    n_words = tableT.shape[1]
    n_blocks = (n_words + NB - 1) // NB
    return pl.pallas_call(
        _detile_body,
        grid=(n_blocks,),
        in_specs=[pl.BlockSpec((EMBED, NB), lambda i: (0, i))],
        out_specs=pl.BlockSpec((NB // 2, 2 * EMBED), lambda i: (i, 0)),
        out_shape=jax.ShapeDtypeStruct(
            (n_blocks * (NB // 2), 2 * EMBED), jnp.float32),
    )(tableT)


def _mlp_body(g2_ref, tag_ref, dep_ref, temb_ref, demb_ref, w1_ref, b1_ref,
              w2_ref, b2_ref, out_ref, pt_ref, pd_ref):
    blk = tag_ref.shape[0]

    @pl.when(pl.program_id(0) == 0)
    def _build_proj():
        for f in range(N_TAG_FEAT):
            base = N_WORD_FEAT * EMBED + f * EMBED
            pt_ref[f * EMBED:(f + 1) * EMBED, :] = jnp.dot(
                temb_ref[...], w1_ref[base:base + EMBED, :],
                preferred_element_type=jnp.float32)
        for f in range(N_DEPREL_FEAT):
            base = (N_WORD_FEAT + N_TAG_FEAT) * EMBED + f * EMBED
            pd_ref[f * EMBED:(f + 1) * EMBED, :] = jnp.dot(
                demb_ref[...], w1_ref[base:base + EMBED, :],
                preferred_element_type=jnp.float32)

    h = jnp.dot(g2_ref[0].astype(jnp.bfloat16),
                w1_ref[0:2 * EMBED, :].astype(jnp.bfloat16),
                preferred_element_type=jnp.float32)
    for j in range(1, NPAIR):
        h = h + jnp.dot(
            g2_ref[j].astype(jnp.bfloat16),
            w1_ref[j * 2 * EMBED:(j + 1) * 2 * EMBED, :].astype(jnp.bfloat16),
            preferred_element_type=jnp.float32)

    tag_ids = tag_ref[...]
    dep_ids = dep_ref[...]
    a_t = jnp.concatenate(
        [jnp.broadcast_to(tag_ids[:, f:f + 1], (blk, EMBED))
         for f in range(N_TAG_FEAT)], axis=1)
    a_d = jnp.concatenate(
        [jnp.broadcast_to(dep_ids[:, f:f + 1], (blk, EMBED))
         for f in range(N_DEPREL_FEAT)], axis=1)
    t_t = lax.rem(lax.broadcasted_iota(jnp.int32, (blk, N_TAG_FEAT * EMBED), 1),
                  EMBED)
    t_d = lax.rem(lax.broadcasted_iota(jnp.int32, (blk, N_DEPREL_FEAT * EMBED), 1),
                  EMBED)
    oh_t = (a_t == t_t).astype(jnp.bfloat16)
    oh_d = (a_d == t_d).astype(jnp.bfloat16)

    h = h + jnp.dot(oh_t, pt_ref[...].astype(jnp.bfloat16),
                    preferred_element_type=jnp.float32)
    h = h + jnp.dot(oh_d, pd_ref[...].astype(jnp.bfloat16),
                    preferred_element_type=jnp.float32)
    h = jnp.maximum(h + b1_ref[...], 0.0)
    out_ref[...] = jnp.dot(h, w2_ref[...],
                           preferred_element_type=jnp.float32) + b2_ref[...]


def _mlp(g2, tag_ids, dep_ids, tag_emb, deprel_emb, W1, b1, W2, b2):
    blk = 512
    grid = (tag_ids.shape[0] // blk,)
    return pl.pallas_call(
        _mlp_body,
        grid=grid,
        in_specs=[
            pl.BlockSpec((NPAIR, blk, 2 * EMBED), lambda i: (0, i, 0)),
            pl.BlockSpec((blk, N_TAG_FEAT), lambda i: (i, 0)),
            pl.BlockSpec((blk, N_DEPREL_FEAT), lambda i: (i, 0)),
            pl.BlockSpec((EMBED, EMBED), lambda i: (0, 0)),
            pl.BlockSpec((EMBED, EMBED), lambda i: (0, 0)),
            pl.BlockSpec((W1.shape[0], HIDDEN), lambda i: (0, 0)),
            pl.BlockSpec((1, HIDDEN), lambda i: (0, 0)),
            pl.BlockSpec((HIDDEN, N_CLASSES), lambda i: (0, 0)),
            pl.BlockSpec((1, N_CLASSES), lambda i: (0, 0)),
        ],
        out_specs=pl.BlockSpec((blk, N_CLASSES), lambda i: (i, 0)),
        out_shape=jax.ShapeDtypeStruct((tag_ids.shape[0], N_CLASSES),
                                       jnp.float32),
        scratch_shapes=[
            pltpu.VMEM((N_TAG_FEAT * EMBED, HIDDEN), jnp.float32),
            pltpu.VMEM((N_DEPREL_FEAT * EMBED, HIDDEN), jnp.float32),
        ],
    )(g2, tag_ids, dep_ids, tag_emb, deprel_emb, W1, b1, W2, b2)


def kernel(word_id_batch, tag_id_batch, deprel_id_batch, word_emb, tag_emb,
           deprel_emb, W1, b1, W2, b2):
    table_lin = _detile(jnp.swapaxes(word_emb, 0, 1))
    table_sc = table_lin.reshape(table_lin.shape[0] * 2, EMBED)
    b1r = b1.reshape(1, HIDDEN)
    b2r = b2.reshape(1, N_CLASSES)
    outs = []
    for s in range(N_SLICES):
        sl = slice(s * BS, (s + 1) * BS)
        g2 = _sc_gather(table_sc, word_id_batch[sl])
        outs.append(_mlp(g2, tag_id_batch[sl], deprel_id_batch[sl], tag_emb,
                         deprel_emb, W1, b1r, W2, b2r))
    return jnp.concatenate(outs, axis=0)
